# Initial kernel scaffold; baseline (speedup 1.0000x reference)
import jax
import jax.numpy as jnp
from jax.experimental import pallas as pl


def kernel(seq_len, pos_embedding):
    raise NotImplementedError("write your pallas kernel here")


# speedup vs baseline: 1.4737x; 1.4737x over previous
import functools

import jax
import jax.numpy as jnp
from jax import lax
from jax.experimental import pallas as pl
from jax.experimental.pallas import tpu as pltpu
from jax.experimental.pallas import tpu_sc as plsc

MAX_LEN = 8192
HIDDEN_DIM = 1024

_INFO = plsc.get_sparse_core_info()
_NC = _INFO.num_cores
_NS = _INFO.num_subcores
_L = _INFO.num_lanes
_NW = _NC * _NS
_B_PER_W = MAX_LEN // _NW
_CHUNK = 64
_NCHUNK = _B_PER_W // _CHUNK


def _pos_encoding_kernel(limit_hbm, table_hbm, out_hbm, limit_v, idx_v,
                         rows_v, sem):
    wid = lax.axis_index("s") * _NC + lax.axis_index("c")
    base = wid * _B_PER_W

    pltpu.sync_copy(limit_hbm, limit_v)
    lim = limit_v[...]

    for c in range(_NCHUNK):
        for j in range(_CHUNK // _L):
            off = c * _CHUNK + j * _L
            vec = lax.iota(jnp.int32, _L) + (base + off)
            idx_v[c, pl.ds(j * _L, _L)] = jnp.minimum(vec, lim)

    for c in range(_NCHUNK):
        pltpu.async_copy(table_hbm.at[idx_v.at[c]], rows_v, sem).wait()
        pltpu.sync_copy(rows_v, out_hbm.at[pl.ds(base + c * _CHUNK, _CHUNK)])


@functools.partial(jax.jit, static_argnums=())
def _run(limit, table):
    kern = functools.partial(
        pl.kernel,
        mesh=plsc.VectorSubcoreMesh(core_axis_name="c", subcore_axis_name="s"),
        out_type=jax.ShapeDtypeStruct((MAX_LEN, HIDDEN_DIM), jnp.float32),
        scratch_types=[
            pltpu.VMEM((_L,), jnp.int32),
            pltpu.VMEM((_NCHUNK, _CHUNK), jnp.int32),
            pltpu.VMEM((_CHUNK, HIDDEN_DIM), jnp.float32),
            pltpu.SemaphoreType.DMA,
        ],
    )(_pos_encoding_kernel)
    return kern(limit, table)


def kernel(seq_len, pos_embedding):
    limit = jnp.full((_L,), jnp.int32(seq_len) - 1, dtype=jnp.int32)
    return _run(limit, pos_embedding)


# speedup vs baseline: 1.5283x; 1.0371x over previous
import functools

import jax
import jax.numpy as jnp
from jax import lax
from jax.experimental import pallas as pl
from jax.experimental.pallas import tpu as pltpu
from jax.experimental.pallas import tpu_sc as plsc

MAX_LEN = 8192
HIDDEN_DIM = 1024

_INFO = plsc.get_sparse_core_info()
_NC = _INFO.num_cores
_NS = _INFO.num_subcores
_L = _INFO.num_lanes
_NW = _NC * _NS
_B_PER_W = MAX_LEN // _NW
_CHUNK = 32
_NCHUNK = _B_PER_W // _CHUNK
_NBUF = 3


def _pos_encoding_kernel(limit_hbm, table_hbm, out_hbm, limit_v, idx_v,
                         rows_v, *sems):
    gsems, ssems = sems[:_NBUF], sems[_NBUF:]
    wid = lax.axis_index("s") * _NC + lax.axis_index("c")
    base = wid * _B_PER_W

    pltpu.sync_copy(limit_hbm, limit_v)
    lim = limit_v[...]

    for c in range(_NCHUNK):
        for j in range(_CHUNK // _L):
            off = c * _CHUNK + j * _L
            vec = lax.iota(jnp.int32, _L) + (base + off)
            idx_v[c, pl.ds(j * _L, _L)] = jnp.minimum(vec, lim)

    def gather(c):
        return pltpu.async_copy(table_hbm.at[idx_v.at[c]],
                                rows_v.at[c % _NBUF], gsems[c % _NBUF])

    gh = [None] * _NCHUNK
    sh = [None] * _NCHUNK
    for c in range(_NBUF):
        gh[c] = gather(c)
    for c in range(_NCHUNK):
        gh[c].wait()
        sh[c] = pltpu.async_copy(
            rows_v.at[c % _NBUF],
            out_hbm.at[pl.ds(base + c * _CHUNK, _CHUNK)], ssems[c % _NBUF])
        if c + _NBUF < _NCHUNK:
            sh[c].wait()
            gh[c + _NBUF] = gather(c + _NBUF)
    for c in range(_NCHUNK - _NBUF, _NCHUNK):
        sh[c].wait()


@functools.partial(jax.jit, static_argnums=())
def _run(limit, table):
    kern = functools.partial(
        pl.kernel,
        mesh=plsc.VectorSubcoreMesh(core_axis_name="c", subcore_axis_name="s"),
        out_type=jax.ShapeDtypeStruct((MAX_LEN, HIDDEN_DIM), jnp.float32),
        scratch_types=[
            pltpu.VMEM((_L,), jnp.int32),
            pltpu.VMEM((_NCHUNK, _CHUNK), jnp.int32),
            pltpu.VMEM((_NBUF, _CHUNK, HIDDEN_DIM), jnp.float32),
        ] + [pltpu.SemaphoreType.DMA] * (2 * _NBUF),
    )(_pos_encoding_kernel)
    return kern(limit, table)


def kernel(seq_len, pos_embedding):
    limit = jnp.full((_L,), jnp.int32(seq_len) - 1, dtype=jnp.int32)
    return _run(limit, pos_embedding)
